# baseline (speedup 1.0000x reference)
import jax
import jax.numpy as jnp
from jax import lax
from jax.experimental import pallas as pl
from jax.experimental.pallas import tpu as pltpu
from jax.experimental.pallas import tpu_sc as plsc

B = 16384
S = 10
F = 32
NC = 2
NS = 16
NW = NC * NS

B_PER_W = B // NW
PAIRS_PER_W = B_PER_W * S
IDXW = 128
IID_ROWS = PAIRS_PER_W // IDXW
UID_ROWS = B_PER_W // IDXW
N_CHUNKS = 4
CHUNK_B = B_PER_W // N_CHUNKS
CHUNK_PAIRS = CHUNK_B * S
CHUNK_IID_ROWS = CHUNK_PAIRS // IDXW
GROUP_B = 8
GROUP_SC = GROUP_B * S
N_GROUPS = CHUNK_B // GROUP_B


def _sc_kernel(iid_hbm, uid_hbm, doc_hbm, usr_hbm, out_hbm,
               iid_v, uid_v, doc_v, usr_v, out_v, sem):
  wid = lax.axis_index("s") * NC + lax.axis_index("c")
  ibase = wid * IID_ROWS
  ubase = wid * UID_ROWS

  pltpu.sync_copy(iid_hbm.at[pl.ds(ibase, IID_ROWS)], iid_v)
  pltpu.sync_copy(uid_hbm.at[pl.ds(ubase, UID_ROWS)], uid_v)

  udescs = []
  for j in range(UID_ROWS):
    udescs.append(pltpu.async_copy(
        usr_hbm.at[uid_v.at[j]], usr_v.at[pl.ds(j * IDXW, IDXW)], sem))
  for d in udescs:
    d.wait()

  zeros16 = jnp.zeros((16,), jnp.float32)

  for c in range(N_CHUNKS):
    descs = []
    for j in range(CHUNK_IID_ROWS):
      descs.append(pltpu.async_copy(
          doc_hbm.at[iid_v.at[c * CHUNK_IID_ROWS + j]],
          doc_v.at[pl.ds(j * IDXW, IDXW)], sem))

    @pl.loop(0, CHUNK_PAIRS // 16)
    def _zero(i):
      out_v[pl.ds(c * CHUNK_PAIRS + i * 16, 16)] = zeros16

    for d in descs:
      d.wait()

    @pl.loop(0, N_GROUPS)
    def _body(i):
      for bb in range(GROUP_B):
        b = i * GROUP_B + bb
        bg = c * CHUNK_B + b
        u0 = usr_v[bg, pl.ds(0, 16)]
        u1 = usr_v[bg, pl.ds(16, 16)]
        base = jnp.full((16,), c * CHUNK_PAIRS + b * S, jnp.int32)
        for s in range(S):
          r = b * S + s
          d0 = doc_v[r, pl.ds(0, 16)]
          d1 = doc_v[r, pl.ds(16, 16)]
          w = d0 * u0 + d1 * u1
          plsc.addupdate_scatter(out_v, [base + s], w)

  pltpu.sync_copy(out_v, out_hbm.at[pl.ds(wid * PAIRS_PER_W, PAIRS_PER_W)])


@jax.jit
def _scores(iid2d, uid2d, docEmbed, userEmbed):
  mesh = plsc.VectorSubcoreMesh(core_axis_name="c", subcore_axis_name="s")
  flat = pl.kernel(
      _sc_kernel,
      out_type=jax.ShapeDtypeStruct((B * S,), jnp.float32),
      mesh=mesh,
      compiler_params=pltpu.CompilerParams(
          needs_layout_passes=False, use_tc_tiling_on_sc=False),
      scratch_types=[
          pltpu.VMEM((IID_ROWS, IDXW), jnp.int32),
          pltpu.VMEM((UID_ROWS, IDXW), jnp.int32),
          pltpu.VMEM((CHUNK_PAIRS, F), jnp.float32),
          pltpu.VMEM((B_PER_W, F), jnp.float32),
          pltpu.VMEM((PAIRS_PER_W,), jnp.float32),
          pltpu.SemaphoreType.DMA,
      ],
  )(iid2d, uid2d, docEmbed, userEmbed)
  return flat.reshape(B, S)


def kernel(item_ids, user_ids, docEmbed, userEmbed):
  iid2d = item_ids.astype(jnp.int32).reshape(NW * IID_ROWS, IDXW)
  uid2d = user_ids.astype(jnp.int32).reshape(NW * UID_ROWS, IDXW)
  return _scores(iid2d, uid2d, docEmbed, userEmbed)


# speedup vs baseline: 1.5340x; 1.5340x over previous
import jax
import jax.numpy as jnp
from jax import lax
from jax.experimental import pallas as pl
from jax.experimental.pallas import tpu as pltpu
from jax.experimental.pallas import tpu_sc as plsc

B = 16384
S = 10
F = 32
NW = 32
B_PER_W = B // NW
PAIRS_PER_W = B_PER_W * S
IDXW = 128
N_STAGES = PAIRS_PER_W // IDXW
UID_ROWS = B_PER_W // IDXW
GROUPS = IDXW // 16


def _compute_stage(buf, st, brow_v, usr_v, out_v):
  for g in range(GROUPS):
    prow = jnp.arange(16, dtype=jnp.int32) + (g * 16)
    urow = brow_v[st, pl.ds(g * 16, 16)]
    acc = jnp.zeros((16,), jnp.float32)
    for f in range(F):
      fv = jnp.full((16,), f, jnp.int32)
      dv = plsc.load_gather(buf, [prow, fv])
      uv = plsc.load_gather(usr_v, [urow, fv])
      acc = acc + dv * uv
    out_v[pl.ds(st * IDXW + g * 16, 16)] = acc


def _sc_kernel(itT_hbm, uid_hbm, doc_hbm, usr_hbm, out_hbm,
               it_v, uid_v, did_v, brow_v, usr_v, doc0, doc1, out_v,
               sem_u, sem_e, sem_o):
  wid = lax.axis_index("s") * 2 + lax.axis_index("c")
  wb = wid * B_PER_W

  pltpu.sync_copy(itT_hbm.at[:, pl.ds(wb, B_PER_W)], it_v)
  pltpu.sync_copy(uid_hbm.at[pl.ds(wb, B_PER_W)], uid_v)

  udescs = []
  for j in range(UID_ROWS):
    udescs.append(pltpu.async_copy(
        usr_hbm.at[uid_v.at[pl.ds(j * IDXW, IDXW)]],
        usr_v.at[pl.ds(j * IDXW, IDXW)], sem_u))

  lane = jnp.arange(16, dtype=jnp.int32)

  @pl.loop(0, N_STAGES)
  def _build(st):
    for g in range(GROUPS):
      pv = lane + (st * IDXW + g * 16)
      bv = (pv * 6554) >> 16
      sv = pv - bv * 10
      ids = plsc.load_gather(it_v, [sv, bv])
      did_v[st, pl.ds(g * 16, 16)] = ids
      brow_v[st, pl.ds(g * 16, 16)] = bv

  pltpu.async_copy(doc_hbm.at[did_v.at[0]], doc0, sem_e)

  for d in udescs:
    d.wait()

  @pl.loop(0, N_STAGES // 2)
  def _body(i):
    s0 = i * 2
    d_odd = pltpu.async_copy(doc_hbm.at[did_v.at[s0 + 1]], doc1, sem_o)
    pltpu.make_async_copy(doc_hbm.at[did_v.at[s0]], doc0, sem_e).wait()
    _compute_stage(doc0, s0, brow_v, usr_v, out_v)

    @pl.when(i < N_STAGES // 2 - 1)
    def _fire_even():
      pltpu.async_copy(doc_hbm.at[did_v.at[s0 + 2]], doc0, sem_e)

    d_odd.wait()
    _compute_stage(doc1, s0 + 1, brow_v, usr_v, out_v)

  pltpu.sync_copy(out_v, out_hbm.at[pl.ds(wid * PAIRS_PER_W, PAIRS_PER_W)])


@jax.jit
def _scores(itT, uid, docEmbed, userEmbed):
  mesh = plsc.VectorSubcoreMesh(core_axis_name="c", subcore_axis_name="s")
  flat = pl.kernel(
      _sc_kernel,
      out_type=jax.ShapeDtypeStruct((B * S,), jnp.float32),
      mesh=mesh,
      compiler_params=pltpu.CompilerParams(
          needs_layout_passes=False, use_tc_tiling_on_sc=False),
      scratch_types=[
          pltpu.VMEM((S, B_PER_W), jnp.int32),
          pltpu.VMEM((B_PER_W,), jnp.int32),
          pltpu.VMEM((N_STAGES, IDXW), jnp.int32),
          pltpu.VMEM((N_STAGES, IDXW), jnp.int32),
          pltpu.VMEM((B_PER_W, F), jnp.float32),
          pltpu.VMEM((IDXW, F), jnp.float32),
          pltpu.VMEM((IDXW, F), jnp.float32),
          pltpu.VMEM((PAIRS_PER_W,), jnp.float32),
          pltpu.SemaphoreType.DMA,
          pltpu.SemaphoreType.DMA,
          pltpu.SemaphoreType.DMA,
      ],
  )(itT, uid, docEmbed, userEmbed)
  return flat.reshape(B, S)


def kernel(item_ids, user_ids, docEmbed, userEmbed):
  itT = item_ids.astype(jnp.int32).T
  uid = user_ids.astype(jnp.int32)
  return _scores(itT, uid, docEmbed, userEmbed)
